# baseline (speedup 1.0000x reference)
import jax
import jax.numpy as jnp
from jax.experimental import pallas as pl
from jax.experimental.pallas import tpu as pltpu

_H, _A, _E = 64, 5, 3
_BB = 16384
_NROW = _E * _A + 1


def _fused_body(xt_ref, act_ref, w1t_ref, b1_ref, w2_ref, b2_ref, whp_ref,
                bh_ref, wc1t_ref, bc1_ref, wc2_ref, bc2_ref, wc3t_ref,
                bc3_ref, act_out_ref, logp_ref, ent_ref, val_ref,
                s_w1t, s_b1, s_w2, s_b2, s_whp, s_bh, s_wc1t, s_bc1, s_wc2,
                s_bc2, s_wc3t, s_bc3, ws, sems):
    bb = xt_ref.shape[1]

    @pl.when(pl.program_id(0) == 0)
    def _pack():
        pairs = (
            (w1t_ref, s_w1t), (b1_ref, s_b1), (w2_ref, s_w2), (b2_ref, s_b2),
            (whp_ref, s_whp), (bh_ref, s_bh), (wc1t_ref, s_wc1t),
            (bc1_ref, s_bc1), (wc2_ref, s_wc2), (bc2_ref, s_bc2),
            (wc3t_ref, s_wc3t), (bc3_ref, s_bc3),
        )
        for k, (src, dst) in enumerate(pairs):
            pltpu.make_async_copy(src, dst, sems.at[k]).start()
        for k, (src, dst) in enumerate(pairs):
            pltpu.make_async_copy(src, dst, sems.at[k]).wait()

        z = jnp.zeros((_H, _H), jnp.float32)
        ws[0:_H, 0:194] = s_w1t[...]
        ws[_H:2 * _H, 0:194] = s_wc1t[...]
        ws[0:_H, 256:320] = s_w2[...].T
        ws[0:_H, 320:384] = z
        ws[_H:2 * _H, 256:320] = z
        ws[_H:2 * _H, 320:384] = s_wc2[...].T
        whp = s_whp[...]
        for e in range(_E):
            ws[_A * e:_A * e + _A, 384:448] = whp[:, e, :]
            ws[_A * e:_A * e + _A, 768:769] = s_bh[e:e + 1, :].T
        ws[0:_E * _A, 448:512] = jnp.zeros((_E * _A, _H), jnp.float32)
        ws[_E * _A:_NROW, 384:448] = jnp.zeros((1, _H), jnp.float32)
        ws[_E * _A:_NROW, 448:512] = s_wc3t[...]
        ws[0:_H, 512:513] = s_b1[...].T
        ws[_H:2 * _H, 512:513] = s_bc1[...].T
        ws[0:_H, 640:641] = s_b2[...].T
        ws[_H:2 * _H, 640:641] = s_bc2[...].T
        ws[_E * _A:_NROW, 768:769] = s_bc3[...]

    xt = xt_ref[...]
    h = jnp.tanh(jnp.dot(ws[:, 0:194], xt,
                         preferred_element_type=jnp.float32) + ws[:, 512:513])
    g = jnp.tanh(jnp.dot(ws[:, 256:384], h,
                         preferred_element_type=jnp.float32) + ws[:, 640:641])
    o = (jnp.dot(ws[0:_NROW, 384:512], g, preferred_element_type=jnp.float32)
         + ws[0:_NROW, 768:769])

    x0, x1, x2 = xt[0:1, :], xt[1:2, :], xt[2:3, :]
    is0 = (x0 >= x1) & (x0 >= x2)
    is1 = jnp.logical_not(is0) & (x1 >= x2)
    ev5 = jnp.where(is0, 0, jnp.where(is1, _A, 2 * _A))

    act = act_ref[...]
    ki = jax.lax.broadcasted_iota(jnp.int32, (_NROW, bb), 0)
    msel = (ki >= ev5) & (ki < ev5 + _A)
    mact = ki == ev5 + act.reshape(1, bb)

    eo = jnp.exp(o)
    z1 = jnp.where(msel, eo, 0.0)
    se = jnp.sum(z1, axis=0, keepdims=True)
    sl = jnp.sum(z1 * o, axis=0, keepdims=True)
    la = jnp.sum(jnp.where(mact, o, 0.0), axis=0, keepdims=True)
    lse = jnp.log(se)

    act_out_ref[...] = act
    logp_ref[...] = (la - lse).reshape(bb)
    ent_ref[...] = (lse - sl / se).reshape(bb)
    val_ref[...] = o[_E * _A:_E * _A + 1, :].reshape(bb)


def kernel(x, action, W1, b1, W2, b2, Wh, bh, Wc1, bc1, Wc2, bc2, Wc3, bc3):
    B, OBS = x.shape
    H, A, E = _H, _A, _E

    xt = x.T
    act = action.astype(jnp.int32)

    nb = B // _BB
    anyspec = pl.BlockSpec(memory_space=pl.ANY)
    out_shape = [
        jax.ShapeDtypeStruct((B,), jnp.int32),
        jax.ShapeDtypeStruct((B,), jnp.float32),
        jax.ShapeDtypeStruct((B,), jnp.float32),
        jax.ShapeDtypeStruct((B,), jnp.float32),
    ]
    act_out, logp, ent, val = pl.pallas_call(
        _fused_body,
        grid=(nb,),
        in_specs=[
            pl.BlockSpec((OBS, _BB), lambda i: (0, i)),
            pl.BlockSpec((_BB,), lambda i: (i,)),
        ] + [anyspec] * 12,
        out_specs=[pl.BlockSpec((_BB,), lambda i: (i,))] * 4,
        out_shape=out_shape,
        scratch_shapes=[
            pltpu.VMEM((H, OBS), jnp.float32),
            pltpu.VMEM((1, H), jnp.float32),
            pltpu.VMEM((H, H), jnp.float32),
            pltpu.VMEM((1, H), jnp.float32),
            pltpu.VMEM((A, E, H), jnp.float32),
            pltpu.VMEM((E, A), jnp.float32),
            pltpu.VMEM((H, OBS), jnp.float32),
            pltpu.VMEM((1, H), jnp.float32),
            pltpu.VMEM((H, H), jnp.float32),
            pltpu.VMEM((1, H), jnp.float32),
            pltpu.VMEM((1, H), jnp.float32),
            pltpu.VMEM((1, 1), jnp.float32),
            pltpu.VMEM((2 * H, 1024), jnp.float32),
            pltpu.SemaphoreType.DMA((12,)),
        ],
        compiler_params=pltpu.CompilerParams(
            dimension_semantics=("arbitrary",),
            vmem_limit_bytes=56 * 1024 * 1024,
        ),
        name="agent_fused_t",
    )(xt, act, W1.T, b1.reshape(1, H), W2, b2.reshape(1, H),
      Wh.transpose(2, 0, 1), bh, Wc1.T, bc1.reshape(1, H), Wc2,
      bc2.reshape(1, H), Wc3.T, bc3.reshape(1, 1))

    return act_out, logp, ent, val.reshape(B, 1)


# speedup vs baseline: 1.0901x; 1.0901x over previous
import jax
import jax.numpy as jnp
from jax.experimental import pallas as pl
from jax.experimental.pallas import tpu as pltpu

_H, _A, _E = 64, 5, 3
_BB = 16384
_NROW = _E * _A + 1


def _fused_body(xt_ref, act_ref, w1t_ref, b1_ref, w2_ref, b2_ref, whp_ref,
                bh_ref, wc1t_ref, bc1_ref, wc2_ref, bc2_ref, wc3t_ref,
                bc3_ref, act_out_ref, logp_ref, ent_ref, val_ref, ws):
    bb = xt_ref.shape[1]

    @pl.when(pl.program_id(0) == 0)
    def _pack():
        z = jnp.zeros((_H, _H), jnp.float32)
        ws[0:_H, 0:194] = w1t_ref[...]
        ws[_H:2 * _H, 0:194] = wc1t_ref[...]
        ws[0:_H, 256:320] = w2_ref[...].T
        ws[0:_H, 320:384] = z
        ws[_H:2 * _H, 256:320] = z
        ws[_H:2 * _H, 320:384] = wc2_ref[...].T
        whp = whp_ref[...]
        for e in range(_E):
            ws[_A * e:_A * e + _A, 384:448] = whp[:, e, :]
            ws[_A * e:_A * e + _A, 768:769] = bh_ref[e:e + 1, :].T
        ws[0:_E * _A, 448:512] = jnp.zeros((_E * _A, _H), jnp.float32)
        ws[_E * _A:_NROW, 384:448] = jnp.zeros((1, _H), jnp.float32)
        ws[_E * _A:_NROW, 448:512] = wc3t_ref[...]
        ws[0:_H, 512:513] = b1_ref[...].T
        ws[_H:2 * _H, 512:513] = bc1_ref[...].T
        ws[0:_H, 640:641] = b2_ref[...].T
        ws[_H:2 * _H, 640:641] = bc2_ref[...].T
        ws[_E * _A:_NROW, 768:769] = bc3_ref[...]

    xt = xt_ref[...]
    h = jnp.tanh(jnp.dot(ws[:, 0:194], xt,
                         preferred_element_type=jnp.float32) + ws[:, 512:513])
    g = jnp.tanh(jnp.dot(ws[:, 256:384], h,
                         preferred_element_type=jnp.float32) + ws[:, 640:641])
    o = (jnp.dot(ws[0:_NROW, 384:512], g, preferred_element_type=jnp.float32)
         + ws[0:_NROW, 768:769])

    x0, x1, x2 = xt[0:1, :], xt[1:2, :], xt[2:3, :]
    is0 = (x0 >= x1) & (x0 >= x2)
    is1 = jnp.logical_not(is0) & (x1 >= x2)
    ev5 = jnp.where(is0, 0, jnp.where(is1, _A, 2 * _A))

    act = act_ref[...]
    ki = jax.lax.broadcasted_iota(jnp.int32, (_NROW, bb), 0)
    msel = (ki >= ev5) & (ki < ev5 + _A)
    mact = ki == ev5 + act.reshape(1, bb)

    eo = jnp.exp(o)
    z1 = jnp.where(msel, eo, 0.0)
    se = jnp.sum(z1, axis=0, keepdims=True)
    sl = jnp.sum(z1 * o, axis=0, keepdims=True)
    la = jnp.sum(jnp.where(mact, o, 0.0), axis=0, keepdims=True)
    lse = jnp.log(se)

    act_out_ref[...] = act
    logp_ref[...] = (la - lse).reshape(bb)
    ent_ref[...] = (lse - sl / se).reshape(bb)
    val_ref[...] = o[_E * _A:_E * _A + 1, :].reshape(bb)


def kernel(x, action, W1, b1, W2, b2, Wh, bh, Wc1, bc1, Wc2, bc2, Wc3, bc3):
    B, OBS = x.shape
    H, A, E = _H, _A, _E

    xt = x.T
    act = action.astype(jnp.int32)

    nb = B // _BB
    full = lambda *dims: (lambda i: tuple(0 for _ in dims))
    out_shape = [
        jax.ShapeDtypeStruct((B,), jnp.int32),
        jax.ShapeDtypeStruct((B,), jnp.float32),
        jax.ShapeDtypeStruct((B,), jnp.float32),
        jax.ShapeDtypeStruct((B,), jnp.float32),
    ]
    act_out, logp, ent, val = pl.pallas_call(
        _fused_body,
        grid=(nb,),
        in_specs=[
            pl.BlockSpec((OBS, _BB), lambda i: (0, i)),
            pl.BlockSpec((_BB,), lambda i: (i,)),
            pl.BlockSpec((H, OBS), full(0, 0)),
            pl.BlockSpec((1, H), full(0, 0)),
            pl.BlockSpec((H, H), full(0, 0)),
            pl.BlockSpec((1, H), full(0, 0)),
            pl.BlockSpec((A, E, H), full(0, 0, 0)),
            pl.BlockSpec((E, A), full(0, 0)),
            pl.BlockSpec((H, OBS), full(0, 0)),
            pl.BlockSpec((1, H), full(0, 0)),
            pl.BlockSpec((H, H), full(0, 0)),
            pl.BlockSpec((1, H), full(0, 0)),
            pl.BlockSpec((1, H), full(0, 0)),
            pl.BlockSpec((1, 1), full(0, 0)),
        ],
        out_specs=[pl.BlockSpec((_BB,), lambda i: (i,))] * 4,
        out_shape=out_shape,
        scratch_shapes=[pltpu.VMEM((2 * H, 1024), jnp.float32)],
        compiler_params=pltpu.CompilerParams(
            dimension_semantics=("arbitrary",),
            vmem_limit_bytes=56 * 1024 * 1024,
        ),
        name="agent_fused_t",
    )(xt, act, W1.T, b1.reshape(1, H), W2, b2.reshape(1, H),
      Wh.transpose(2, 0, 1), bh, Wc1.T, bc1.reshape(1, H), Wc2,
      bc2.reshape(1, H), Wc3.T, bc3.reshape(1, 1))

    return act_out, logp, ent, val.reshape(B, 1)
